# baseline (speedup 1.0000x reference)
import functools

import jax
import jax.numpy as jnp
from jax import lax
from jax.experimental import pallas as pl
from jax.experimental.pallas import tpu as pltpu
from jax.experimental.pallas import tpu_sc as plsc

B = 32
IN_SCALE = 7
D = IN_SCALE * IN_SCALE
BITS = 32
HDC = 512
NNZ = D * HDC // 2
IND_ROWS = 64
IND_PAD = IND_ROWS * HDC
LANES = 16
NUM_CORES = 2
NUM_SUBCORES = 16
NW = NUM_CORES * NUM_SUBCORES
CHUNK = IND_PAD // NW
G = 49


def _sc_scatter_body(idx_hbm, out_hbm, idx_v, chunk_v):
    wid = lax.axis_index("s") * NUM_CORES + lax.axis_index("c")
    base = wid * CHUNK

    z16 = jnp.zeros((LANES,), jnp.int32)

    def zero_body(i, carry):
        chunk_v[pl.ds(i * LANES, LANES)] = z16
        return carry

    lax.fori_loop(0, CHUNK // LANES, zero_body, 0)

    pltpu.sync_copy(idx_hbm, idx_v)

    ones16 = jnp.ones((LANES,), jnp.int32)

    def scat_body(i, carry):
        v = idx_v[pl.ds(i * LANES, LANES)]
        local = v - base
        m = (local >= 0) & (local < CHUNK)
        lc = jnp.clip(local, 0, CHUNK - 1)
        plsc.store_scatter(chunk_v, [lc], ones16, mask=m)
        return carry

    lax.fori_loop(0, NNZ // LANES, scat_body, 0)

    pltpu.sync_copy(chunk_v, out_hbm.at[pl.ds(base, CHUNK)])


@functools.cache
def _make_sc_scatter():
    return functools.partial(
        pl.kernel,
        mesh=plsc.VectorSubcoreMesh(
            core_axis_name="c", subcore_axis_name="s",
            num_cores=NUM_CORES, num_subcores=NUM_SUBCORES,
        ),
        out_type=jax.ShapeDtypeStruct((IND_PAD,), jnp.int32),
        scratch_types=[
            pltpu.VMEM((NNZ,), jnp.int32),
            pltpu.VMEM((CHUNK,), jnp.int32),
        ],
        compiler_params=pltpu.CompilerParams(needs_layout_passes=False),
    )(_sc_scatter_body)


BB = 2


def _tc_body(xi_ref, st_ref, par_ref, ind_ref, sf_ref, proj_ref, rbin_ref):
    blk = pl.program_id(0)

    @pl.when(blk == 0)
    def _():
        sf_ref[...] = st_ref[...] * par_ref[...]

    use_ind = blk == 0
    bit_iota = lax.broadcasted_iota(jnp.int32, (BITS, HDC), 0)
    ztile = jnp.zeros((BITS, HDC), jnp.int32)

    for bb in range(BB):
        for j in range(D):
            x = xi_ref[0, 0, bb, j]
            xb = jnp.bitwise_and(jnp.right_shift(x, bit_iota), 1)
            if bb == 0 and j < 2:
                ptile = jnp.where(use_ind,
                                  ind_ref[j * BITS:(j + 1) * BITS, :], ztile)
                proj_ref[bb, j] = ptile
                rbin_ref[bb, j] = jnp.bitwise_xor(xb, ptile)
            else:
                proj_ref[bb, j] = ztile
                rbin_ref[bb, j] = xb


_tc_call = pl.pallas_call(
    _tc_body,
    grid=(B // BB,),
    in_specs=[
        pl.BlockSpec((1, 1, BB, D), lambda b: (b, 0, 0, 0),
                     memory_space=pltpu.SMEM),
        pl.BlockSpec((B, D), lambda b: (0, 0)),
        pl.BlockSpec((1, D), lambda b: (0, 0)),
        pl.BlockSpec((IND_ROWS, HDC), lambda b: (0, 0)),
    ],
    out_specs=[
        pl.BlockSpec((B, D), lambda b: (0, 0)),
        pl.BlockSpec((BB, D, BITS, HDC), lambda b: (b, 0, 0, 0)),
        pl.BlockSpec((BB, D, BITS, HDC), lambda b: (b, 0, 0, 0)),
    ],
    out_shape=[
        jax.ShapeDtypeStruct((B, D), jnp.float32),
        jax.ShapeDtypeStruct((B, D, BITS, HDC), jnp.int32),
        jax.ShapeDtypeStruct((B, D, BITS, HDC), jnp.int32),
    ],
)


def kernel(data_input, structure_input, meta_input_h1, meta_input_h2,
           meta_input_h3, meta_input_h4, meta_input_h5, noise_var_in_binary,
           fmot_in_binary, meta_output_h1, meta_output_h2, meta_output_h3,
           meta_output_h4, meta_output_h5, noise_var_out, non_zero_indices,
           parameters_temp):
    r = data_input[:, 0:IN_SCALE, :]
    g = data_input[:, IN_SCALE:2 * IN_SCALE, :]
    bch = data_input[:, 2 * IN_SCALE:3 * IN_SCALE, :]
    a = data_input[:, 3 * IN_SCALE:4 * IN_SCALE, :]
    rf = r.reshape(B, D)
    gf = g.reshape(B, D)
    bf = bch.reshape(B, D)
    af = a.reshape(B, D)

    xi = lax.bitcast_convert_type(rf, jnp.int32).reshape(B // BB, 1, BB, D)
    st = structure_input.reshape(B, D)
    par = parameters_temp.reshape(1, D)

    ind = _make_sc_scatter()(non_zero_indices)
    ind2d = ind.reshape(IND_ROWS, HDC)

    sf, proj, r_bin = _tc_call(xi, st, par, ind2d)

    s = sf.reshape(B, IN_SCALE, IN_SCALE)
    deepS = (r, g, bch, a, s)
    return (rf, gf, bf, af, sf, deepS, proj, r_bin)


# speedup vs baseline: 1.0124x; 1.0124x over previous
import functools

import jax
import jax.numpy as jnp
from jax import lax
from jax.experimental import pallas as pl
from jax.experimental.pallas import tpu as pltpu
from jax.experimental.pallas import tpu_sc as plsc

B = 32
IN_SCALE = 7
D = IN_SCALE * IN_SCALE
BITS = 32
HDC = 512
NNZ = D * HDC // 2
IND_ROWS = 64
IND_PAD = IND_ROWS * HDC
LANES = 16
NUM_CORES = 2
NUM_SUBCORES = 16
NW = NUM_CORES * NUM_SUBCORES
CHUNK = IND_PAD // NW
G = 49


def _sc_scatter_body(idx_hbm, out_hbm, idx_v, chunk_v):
    wid = lax.axis_index("s") * NUM_CORES + lax.axis_index("c")
    base = wid * CHUNK

    z16 = jnp.zeros((LANES,), jnp.int32)

    def zero_body(i, carry):
        chunk_v[pl.ds(i * LANES, LANES)] = z16
        return carry

    lax.fori_loop(0, CHUNK // LANES, zero_body, 0)

    pltpu.sync_copy(idx_hbm, idx_v)

    ones16 = jnp.ones((LANES,), jnp.int32)

    def scat_body(i, carry):
        v = idx_v[pl.ds(i * LANES, LANES)]
        local = v - base
        m = (local >= 0) & (local < CHUNK)
        lc = jnp.clip(local, 0, CHUNK - 1)
        plsc.store_scatter(chunk_v, [lc], ones16, mask=m)
        return carry

    lax.fori_loop(0, NNZ // LANES, scat_body, 0)

    pltpu.sync_copy(chunk_v, out_hbm.at[pl.ds(base, CHUNK)])


@functools.cache
def _make_sc_scatter():
    return functools.partial(
        pl.kernel,
        mesh=plsc.VectorSubcoreMesh(
            core_axis_name="c", subcore_axis_name="s",
            num_cores=NUM_CORES, num_subcores=NUM_SUBCORES,
        ),
        out_type=jax.ShapeDtypeStruct((IND_PAD,), jnp.int32),
        scratch_types=[
            pltpu.VMEM((NNZ,), jnp.int32),
            pltpu.VMEM((CHUNK,), jnp.int32),
        ],
        compiler_params=pltpu.CompilerParams(needs_layout_passes=False),
    )(_sc_scatter_body)


BB = 1


def _tc_body(xi_ref, st_ref, par_ref, ind_ref, sf_ref, proj_ref, rbin_ref):
    blk = pl.program_id(0)

    @pl.when(blk == 0)
    def _():
        sf_ref[...] = st_ref[...] * par_ref[...]

    use_ind = blk == 0
    bit_iota = lax.broadcasted_iota(jnp.int32, (BITS, HDC), 0)
    ztile = jnp.zeros((BITS, HDC), jnp.int32)

    for bb in range(BB):
        for j in range(D):
            x = xi_ref[0, 0, bb, j]
            xb = jnp.bitwise_and(jnp.right_shift(x, bit_iota), 1)
            if bb == 0 and j < 2:
                ptile = jnp.where(use_ind,
                                  ind_ref[j * BITS:(j + 1) * BITS, :], ztile)
                proj_ref[bb, j] = ptile
                rbin_ref[bb, j] = jnp.bitwise_xor(xb, ptile)
            else:
                proj_ref[bb, j] = ztile
                rbin_ref[bb, j] = xb


_tc_call = pl.pallas_call(
    _tc_body,
    grid=(B // BB,),
    in_specs=[
        pl.BlockSpec((1, 1, BB, D), lambda b: (b, 0, 0, 0),
                     memory_space=pltpu.SMEM),
        pl.BlockSpec((B, D), lambda b: (0, 0)),
        pl.BlockSpec((1, D), lambda b: (0, 0)),
        pl.BlockSpec((IND_ROWS, HDC), lambda b: (0, 0)),
    ],
    out_specs=[
        pl.BlockSpec((B, D), lambda b: (0, 0)),
        pl.BlockSpec((BB, D, BITS, HDC), lambda b: (b, 0, 0, 0)),
        pl.BlockSpec((BB, D, BITS, HDC), lambda b: (b, 0, 0, 0)),
    ],
    out_shape=[
        jax.ShapeDtypeStruct((B, D), jnp.float32),
        jax.ShapeDtypeStruct((B, D, BITS, HDC), jnp.int32),
        jax.ShapeDtypeStruct((B, D, BITS, HDC), jnp.int32),
    ],
)


def kernel(data_input, structure_input, meta_input_h1, meta_input_h2,
           meta_input_h3, meta_input_h4, meta_input_h5, noise_var_in_binary,
           fmot_in_binary, meta_output_h1, meta_output_h2, meta_output_h3,
           meta_output_h4, meta_output_h5, noise_var_out, non_zero_indices,
           parameters_temp):
    r = data_input[:, 0:IN_SCALE, :]
    g = data_input[:, IN_SCALE:2 * IN_SCALE, :]
    bch = data_input[:, 2 * IN_SCALE:3 * IN_SCALE, :]
    a = data_input[:, 3 * IN_SCALE:4 * IN_SCALE, :]
    rf = r.reshape(B, D)
    gf = g.reshape(B, D)
    bf = bch.reshape(B, D)
    af = a.reshape(B, D)

    xi = lax.bitcast_convert_type(rf, jnp.int32).reshape(B // BB, 1, BB, D)
    st = structure_input.reshape(B, D)
    par = parameters_temp.reshape(1, D)

    ind = _make_sc_scatter()(non_zero_indices)
    ind2d = ind.reshape(IND_ROWS, HDC)

    sf, proj, r_bin = _tc_call(xi, st, par, ind2d)

    s = sf.reshape(B, IN_SCALE, IN_SCALE)
    deepS = (r, g, bch, a, s)
    return (rf, gf, bf, af, sf, deepS, proj, r_bin)
